# Initial kernel scaffold; baseline (speedup 1.0000x reference)
import jax
import jax.numpy as jnp
from jax.experimental import pallas as pl


def kernel(batch, table):
    raise NotImplementedError("write your pallas kernel here")


# speedup vs baseline: 5.7713x; 5.7713x over previous
import functools

import jax
import jax.numpy as jnp
from jax import lax
from jax.experimental import pallas as pl
from jax.experimental.pallas import tpu as pltpu, tpu_sc as plsc

CHUNK = 128


@functools.partial(jax.jit, static_argnames=())
def _lookup(idx, table):
    info = plsc.get_sparse_core_info()
    nc, ns = info.num_cores, info.num_subcores
    nw = nc * ns
    n = idx.shape[0]
    d = table.shape[1]
    per_w = n // nw
    n_chunks = per_w // CHUNK
    idx3 = idx.reshape(nw, n_chunks, CHUNK)

    mesh = plsc.VectorSubcoreMesh(core_axis_name="c", subcore_axis_name="s")

    @functools.partial(
        pl.kernel,
        mesh=mesh,
        out_type=jax.ShapeDtypeStruct((n, d), jnp.float32),
        scratch_types=[
            pltpu.VMEM((n_chunks, CHUNK), jnp.int32),
            pltpu.VMEM((CHUNK, d), jnp.float32),
            pltpu.SemaphoreType.DMA,
        ],
    )
    def gather_kernel(idx_hbm, table_hbm, out_hbm, idx_v, rows_v, sem):
        wid = lax.axis_index("s") * nc + lax.axis_index("c")
        pltpu.sync_copy(idx_hbm.at[wid], idx_v)
        base = wid * per_w

        def body(j, carry):
            pltpu.async_copy(table_hbm.at[idx_v.at[j]], rows_v, sem).wait()
            pltpu.sync_copy(rows_v, out_hbm.at[pl.ds(base + j * CHUNK, CHUNK)])
            return carry

        lax.fori_loop(0, n_chunks, body, 0)

    return gather_kernel(idx3, table)


def kernel(batch, table):
    b, h = batch.shape
    d = table.shape[1]
    idx = batch.reshape(-1).astype(jnp.int32)
    out = _lookup(idx, table)
    return out.reshape(b, h, d)


# speedup vs baseline: 7.3063x; 1.2660x over previous
import functools

import jax
import jax.numpy as jnp
from jax import lax
from jax.experimental import pallas as pl
from jax.experimental.pallas import tpu as pltpu, tpu_sc as plsc

CHUNK = 128


@jax.jit
def _lookup(idx, table):
    info = plsc.get_sparse_core_info()
    nc, ns = info.num_cores, info.num_subcores
    nw = nc * ns
    n = idx.shape[0]
    d = table.shape[1]
    per_w = n // nw
    n_chunks = per_w // CHUNK
    n_pairs = n_chunks // 2
    idx3 = idx.reshape(nw, n_chunks, CHUNK)

    mesh = plsc.VectorSubcoreMesh(core_axis_name="c", subcore_axis_name="s")

    @functools.partial(
        pl.kernel,
        mesh=mesh,
        out_type=jax.ShapeDtypeStruct((n, d), jnp.float32),
        scratch_types=[
            pltpu.VMEM((n_chunks, CHUNK), jnp.int32),
            pltpu.VMEM((CHUNK, d), jnp.float32),
            pltpu.VMEM((CHUNK, d), jnp.float32),
            pltpu.SemaphoreType.DMA,
            pltpu.SemaphoreType.DMA,
            pltpu.SemaphoreType.DMA,
            pltpu.SemaphoreType.DMA,
        ],
    )
    def gather_kernel(idx_hbm, table_hbm, out_hbm, idx_v, buf0, buf1,
                      gs0, gs1, ws0, ws1):
        wid = lax.axis_index("s") * nc + lax.axis_index("c")
        pltpu.sync_copy(idx_hbm.at[wid], idx_v)
        base = wid * per_w

        def gather_start(j, buf, sem):
            pltpu.async_copy(table_hbm.at[idx_v.at[j]], buf, sem)

        def gather_wait(j, buf, sem):
            pltpu.make_async_copy(table_hbm.at[idx_v.at[j]], buf, sem).wait()

        def out_slab(j):
            return out_hbm.at[pl.ds(base + j * CHUNK, CHUNK)]

        def put_start(j, buf, sem):
            pltpu.async_copy(buf, out_slab(j), sem)

        def put_wait(j, buf, sem):
            pltpu.make_async_copy(buf, out_slab(j), sem).wait()

        gather_start(0, buf0, gs0)
        gather_start(1, buf1, gs1)

        def body(i, carry):
            j0 = 2 * i
            j1 = j0 + 1
            gather_wait(j0, buf0, gs0)
            put_start(j0, buf0, ws0)
            gather_wait(j1, buf1, gs1)
            put_start(j1, buf1, ws1)

            @pl.when(i + 1 < n_pairs)
            def _():
                put_wait(j0, buf0, ws0)
                gather_start(j0 + 2, buf0, gs0)
                put_wait(j1, buf1, ws1)
                gather_start(j1 + 2, buf1, gs1)

            return carry

        lax.fori_loop(0, n_pairs, body, 0)

        put_wait(n_chunks - 2, buf0, ws0)
        put_wait(n_chunks - 1, buf1, ws1)

    return gather_kernel(idx3, table)


def kernel(batch, table):
    b, h = batch.shape
    d = table.shape[1]
    idx = batch.reshape(-1).astype(jnp.int32)
    out = _lookup(idx, table)
    return out.reshape(b, h, d)


# speedup vs baseline: 7.8117x; 1.0692x over previous
import functools

import jax
import jax.numpy as jnp
from jax import lax
from jax.experimental import pallas as pl
from jax.experimental.pallas import tpu as pltpu, tpu_sc as plsc

CHUNK = 128
NBUF = 5


@jax.jit
def _lookup(idx, table):
    info = plsc.get_sparse_core_info()
    nc, ns = info.num_cores, info.num_subcores
    nw = nc * ns
    n = idx.shape[0]
    d = table.shape[1]
    per_w = n // nw
    n_chunks = per_w // CHUNK
    n_groups = n_chunks // NBUF
    idx3 = idx.reshape(nw, n_chunks, CHUNK)

    mesh = plsc.VectorSubcoreMesh(core_axis_name="c", subcore_axis_name="s")

    @functools.partial(
        pl.kernel,
        mesh=mesh,
        out_type=jax.ShapeDtypeStruct((n, d), jnp.float32),
        scratch_types=[
            pltpu.VMEM((n_chunks, CHUNK), jnp.int32),
            [pltpu.VMEM((CHUNK, d), jnp.float32) for _ in range(NBUF)],
            [pltpu.SemaphoreType.DMA for _ in range(NBUF)],
            [pltpu.SemaphoreType.DMA for _ in range(NBUF)],
        ],
    )
    def gather_kernel(idx_hbm, table_hbm, out_hbm, idx_v, bufs, gsems, wsems):
        wid = lax.axis_index("s") * nc + lax.axis_index("c")
        pltpu.sync_copy(idx_hbm.at[wid], idx_v)
        base = wid * per_w

        def gather_start(j, b):
            pltpu.async_copy(table_hbm.at[idx_v.at[j]], bufs[b], gsems[b])

        def gather_wait(j, b):
            pltpu.make_async_copy(
                table_hbm.at[idx_v.at[j]], bufs[b], gsems[b]).wait()

        def out_slab(j):
            return out_hbm.at[pl.ds(base + j * CHUNK, CHUNK)]

        def put_start(j, b):
            pltpu.async_copy(bufs[b], out_slab(j), wsems[b])

        def put_wait(j, b):
            pltpu.make_async_copy(bufs[b], out_slab(j), wsems[b]).wait()

        for b in range(NBUF):
            gather_start(b, b)

        def body(i, carry):
            j0 = i * NBUF
            for b in range(NBUF):
                gather_wait(j0 + b, b)
                put_start(j0 + b, b)

            @pl.when(i + 1 < n_groups)
            def _():
                for b in range(NBUF):
                    put_wait(j0 + b, b)
                    gather_start(j0 + NBUF + b, b)

            return carry

        lax.fori_loop(0, n_groups, body, 0)

        for b in range(NBUF):
            put_wait(n_chunks - NBUF + b, b)

    return gather_kernel(idx3, table)


def kernel(batch, table):
    b, h = batch.shape
    d = table.shape[1]
    idx = batch.reshape(-1).astype(jnp.int32)
    out = _lookup(idx, table)
    return out.reshape(b, h, d)


# speedup vs baseline: 7.9662x; 1.0198x over previous
import functools

import jax
import jax.numpy as jnp
from jax import lax
from jax.experimental import pallas as pl
from jax.experimental.pallas import tpu as pltpu, tpu_sc as plsc

CHUNK = 128
NBUF = 5


@jax.jit
def _lookup(idx, table):
    info = plsc.get_sparse_core_info()
    nc, ns = info.num_cores, info.num_subcores
    nw = nc * ns
    n = idx.shape[0]
    d = table.shape[1]
    per_w = n // nw
    n_chunks = per_w // CHUNK
    n_groups = n_chunks // NBUF
    idx3 = idx.reshape(nw, n_chunks, CHUNK)

    mesh = plsc.VectorSubcoreMesh(core_axis_name="c", subcore_axis_name="s")

    @functools.partial(
        pl.kernel,
        mesh=mesh,
        out_type=jax.ShapeDtypeStruct((n, d), jnp.float32),
        scratch_types=[
            pltpu.VMEM((n_chunks, CHUNK), jnp.int32),
            [pltpu.VMEM((CHUNK, d), jnp.float32) for _ in range(NBUF)],
            [pltpu.SemaphoreType.DMA for _ in range(NBUF)],
            [pltpu.SemaphoreType.DMA for _ in range(NBUF)],
        ],
    )
    def gather_kernel(idx_hbm, table_hbm, out_hbm, idx_v, bufs, gsems, wsems):
        wid = lax.axis_index("s") * nc + lax.axis_index("c")
        pltpu.sync_copy(idx_hbm.at[wid], idx_v)
        base = wid * per_w

        def gather_start(j, b):
            pltpu.async_copy(table_hbm.at[idx_v.at[j]], bufs[b], gsems[b])

        def gather_wait(j, b):
            pltpu.make_async_copy(
                table_hbm.at[idx_v.at[j]], bufs[b], gsems[b]).wait()

        def out_slab(j):
            return out_hbm.at[pl.ds(base + j * CHUNK, CHUNK)]

        def put_start(j, b):
            pltpu.async_copy(bufs[b], out_slab(j), wsems[b])

        def put_wait(j, b):
            pltpu.make_async_copy(bufs[b], out_slab(j), wsems[b]).wait()

        LOOK = NBUF - 2

        for b in range(LOOK):
            gather_start(b, b)

        def body(i, carry):
            j0 = i * NBUF
            for b in range(NBUF):
                j = j0 + b
                s = (b + LOOK) % NBUF

                @pl.when(jnp.logical_and(j >= NBUF - LOOK,
                                         j + LOOK < n_chunks))
                def _():
                    put_wait(j - (NBUF - LOOK), s)

                @pl.when(j + LOOK < n_chunks)
                def _():
                    gather_start(j + LOOK, s)

                gather_wait(j, b)
                put_start(j, b)
            return carry

        lax.fori_loop(0, n_groups, body, 0)

        for b in range(NBUF):
            j = n_chunks - NBUF + b
            put_wait(j, j % NBUF)

    return gather_kernel(idx3, table)


def kernel(batch, table):
    b, h = batch.shape
    d = table.shape[1]
    idx = batch.reshape(-1).astype(jnp.int32)
    out = _lookup(idx, table)
    return out.reshape(b, h, d)


# speedup vs baseline: 7.9772x; 1.0014x over previous
import functools

import jax
import jax.numpy as jnp
from jax import lax
from jax.experimental import pallas as pl
from jax.experimental.pallas import tpu as pltpu, tpu_sc as plsc

CHUNK = 128
NBUF = 5


@jax.jit
def _lookup(idx, table):
    info = plsc.get_sparse_core_info()
    nc, ns = info.num_cores, info.num_subcores
    nw = nc * ns
    n = idx.shape[0]
    d = table.shape[1]
    per_w = n // nw
    n_chunks = per_w // CHUNK
    n_groups = n_chunks // NBUF
    idx3 = idx.reshape(nw, n_chunks, CHUNK)

    mesh = plsc.VectorSubcoreMesh(core_axis_name="c", subcore_axis_name="s")

    @functools.partial(
        pl.kernel,
        mesh=mesh,
        out_type=jax.ShapeDtypeStruct((n, d), jnp.float32),
        scratch_types=[
            pltpu.VMEM((n_chunks, CHUNK), jnp.int32),
            [pltpu.VMEM((CHUNK, d), jnp.float32) for _ in range(NBUF)],
            [pltpu.SemaphoreType.DMA for _ in range(NBUF)],
            [pltpu.SemaphoreType.DMA for _ in range(NBUF)],
        ],
    )
    def gather_kernel(idx_hbm, table_hbm, out_hbm, idx_v, bufs, gsems, wsems):
        wid = lax.axis_index("s") * nc + lax.axis_index("c")
        pltpu.sync_copy(idx_hbm.at[wid], idx_v)
        base = wid * per_w

        def gather_start(j, b):
            pltpu.async_copy(table_hbm.at[idx_v.at[j]], bufs[b], gsems[b])

        def gather_wait(j, b):
            pltpu.make_async_copy(
                table_hbm.at[idx_v.at[j]], bufs[b], gsems[b]).wait()

        def out_slab(j):
            return out_hbm.at[pl.ds(base + j * CHUNK, CHUNK)]

        def put_start(j, b):
            pltpu.async_copy(bufs[b], out_slab(j), wsems[b])

        def put_wait(j, b):
            pltpu.make_async_copy(bufs[b], out_slab(j), wsems[b]).wait()

        LOOK = NBUF - 3

        for b in range(LOOK):
            gather_start(b, b)

        def body(i, carry):
            j0 = i * NBUF
            for b in range(NBUF):
                j = j0 + b
                s = (b + LOOK) % NBUF

                @pl.when(jnp.logical_and(j >= NBUF - LOOK,
                                         j + LOOK < n_chunks))
                def _():
                    put_wait(j - (NBUF - LOOK), s)

                @pl.when(j + LOOK < n_chunks)
                def _():
                    gather_start(j + LOOK, s)

                gather_wait(j, b)
                put_start(j, b)
            return carry

        lax.fori_loop(0, n_groups, body, 0)

        for b in range(NBUF):
            j = n_chunks - NBUF + b
            put_wait(j, j % NBUF)

    return gather_kernel(idx3, table)


def kernel(batch, table):
    b, h = batch.shape
    d = table.shape[1]
    idx = batch.reshape(-1).astype(jnp.int32)
    out = _lookup(idx, table)
    return out.reshape(b, h, d)
